# baseline (speedup 1.0000x reference)
import functools

import jax
import jax.numpy as jnp
from jax import lax
from jax.experimental import pallas as pl
from jax.experimental.pallas import tpu as pltpu
from jax.experimental.pallas import tpu_sc as plsc

D_MODEL = 1024
SCALE = 32.0
NC = 2
NS = 16
NW = NC * NS
LANES = 16
C = 16
NIN = 4
NOUT = 2
B_SC = 20480


@functools.partial(jax.jit, static_argnums=(2,))
def _emb(idx, table, B):
    chunks = B // (NW * C)
    mesh = plsc.VectorSubcoreMesh(core_axis_name="c", subcore_axis_name="s")

    @functools.partial(
        pl.kernel,
        out_type=jax.ShapeDtypeStruct((B, D_MODEL), jnp.float32),
        mesh=mesh,
        scratch_types=(
            [pltpu.VMEM((chunks, C), jnp.int32)]
            + [pltpu.VMEM((C, D_MODEL), jnp.float32)] * (NIN + NOUT)
            + [pltpu.SemaphoreType.DMA] * (NIN + NOUT)
        ),
    )
    def emb_kernel(idx_hbm, table_hbm, out_hbm, idx_v, *bufs_and_sems):
        ins = bufs_and_sems[:NIN]
        outs = bufs_and_sems[NIN:NIN + NOUT]
        sis = bufs_and_sems[NIN + NOUT:2 * NIN + NOUT]
        sos = bufs_and_sems[2 * NIN + NOUT:]
        wid = lax.axis_index("s") * NC + lax.axis_index("c")
        base = wid * (chunks * C)
        pltpu.sync_copy(idx_hbm.at[wid], idx_v)
        for b in range(NIN):
            pltpu.async_copy(table_hbm.at[idx_v.at[b]], ins[b], sis[b])

        def outer(jj, carry):
            for u in range(NIN):
                j = NIN * jj + u
                b, ob = u, u % NOUT
                inb, sib = ins[b], sis[b]
                outb, sob = outs[ob], sos[ob]
                pltpu.make_async_copy(table_hbm.at[idx_v.at[j]], inb, sib).wait()

                @pl.when(j >= NOUT)
                def _():
                    pltpu.make_async_copy(
                        outb, out_hbm.at[pl.ds(base, C)], sob).wait()

                def row_body(r, c2):
                    for k in range(D_MODEL // LANES):
                        sl = pl.ds(k * LANES, LANES)
                        outb[r, sl] = inb[r, sl] * SCALE
                    return c2
                lax.fori_loop(0, C, row_body, 0)

                @pl.when(j < chunks - NIN)
                def _():
                    pltpu.async_copy(table_hbm.at[idx_v.at[j + NIN]], inb, sib)

                pltpu.async_copy(outb, out_hbm.at[pl.ds(base + j * C, C)], sob)
            return carry

        lax.fori_loop(0, chunks // NIN, outer, 0)
        for u in range(NOUT):
            j = chunks - NOUT + u
            pltpu.make_async_copy(
                outs[j % NOUT], out_hbm.at[pl.ds(base + j * C, C)],
                sos[j % NOUT]).wait()

    return emb_kernel(idx, table)


def kernel(x, table):
    b, s = x.shape
    B = b * s
    flat = x.reshape(B).astype(jnp.int32)
    idx_sc = flat[:B_SC].reshape(NW, B_SC // (NW * C), C)
    out_sc = _emb(idx_sc, table, B_SC)
    out_tc = jnp.take(table, flat[B_SC:], axis=0) * SCALE
    out = jnp.concatenate([out_sc, out_tc], axis=0)
    return out.reshape(b, s, D_MODEL)


# speedup vs baseline: 1.7263x; 1.7263x over previous
import functools

import jax
import jax.numpy as jnp
from jax import lax
from jax.experimental import pallas as pl
from jax.experimental.pallas import tpu as pltpu
from jax.experimental.pallas import tpu_sc as plsc

D_MODEL = 1024
SCALE = 32.0
NC = 2
NS = 16
NW = NC * NS
LANES = 16
C = 16
NIN = 4
NOUT = 2


@functools.partial(jax.jit, static_argnums=(2,))
def _emb(idx, table, B):
    chunks = B // (NW * C)
    mesh = plsc.VectorSubcoreMesh(core_axis_name="c", subcore_axis_name="s")

    @functools.partial(
        pl.kernel,
        out_type=jax.ShapeDtypeStruct((B, D_MODEL), jnp.float32),
        mesh=mesh,
        scratch_types=(
            [pltpu.VMEM((chunks, C), jnp.int32)]
            + [pltpu.VMEM((C, D_MODEL), jnp.float32)] * (NIN + NOUT)
            + [pltpu.SemaphoreType.DMA] * (NIN + NOUT)
        ),
    )
    def emb_kernel(idx_hbm, table_hbm, out_hbm, idx_v, *bufs_and_sems):
        ins = bufs_and_sems[:NIN]
        outs = bufs_and_sems[NIN:NIN + NOUT]
        sis = bufs_and_sems[NIN + NOUT:2 * NIN + NOUT]
        sos = bufs_and_sems[2 * NIN + NOUT:]
        wid = lax.axis_index("s") * NC + lax.axis_index("c")
        base = wid * (chunks * C)
        pltpu.sync_copy(idx_hbm.at[wid], idx_v)
        for b in range(NIN):
            pltpu.async_copy(table_hbm.at[idx_v.at[b]], ins[b], sis[b])

        def outer(jj, carry):
            for u in range(NIN):
                j = NIN * jj + u
                b, ob = u, u % NOUT
                inb, sib = ins[b], sis[b]
                outb, sob = outs[ob], sos[ob]
                pltpu.make_async_copy(table_hbm.at[idx_v.at[j]], inb, sib).wait()

                @pl.when(j >= NOUT)
                def _():
                    pltpu.make_async_copy(
                        outb, out_hbm.at[pl.ds(base, C)], sob).wait()

                def row_body(r, c2):
                    for rr in range(2):
                        for k in range(D_MODEL // LANES):
                            sl = pl.ds(k * LANES, LANES)
                            outb[2 * r + rr, sl] = inb[2 * r + rr, sl] * SCALE
                    return c2
                lax.fori_loop(0, C // 2, row_body, 0)

                @pl.when(j < chunks - NIN)
                def _():
                    pltpu.async_copy(table_hbm.at[idx_v.at[j + NIN]], inb, sib)

                pltpu.async_copy(outb, out_hbm.at[pl.ds(base + j * C, C)], sob)
            return carry

        lax.fori_loop(0, chunks // NIN, outer, 0)
        for u in range(NOUT):
            j = chunks - NOUT + u
            pltpu.make_async_copy(
                outs[j % NOUT], out_hbm.at[pl.ds(base + j * C, C)],
                sos[j % NOUT]).wait()

    return emb_kernel(idx, table)


def kernel(x, table):
    b, s = x.shape
    B = b * s
    idx = x.reshape(NW, B // (NW * C), C).astype(jnp.int32)
    out = _emb(idx, table, B)
    return out.reshape(b, s, D_MODEL)


# speedup vs baseline: 2.0379x; 1.1805x over previous
import functools

import jax
import jax.numpy as jnp
from jax import lax
from jax.experimental import pallas as pl
from jax.experimental.pallas import tpu as pltpu
from jax.experimental.pallas import tpu_sc as plsc

D_MODEL = 1024
SCALE = 32.0
NC = 2
NS = 16
NW = NC * NS
LANES = 16
C = 16
NIN = 4
NOUT = 2


@functools.partial(jax.jit, static_argnums=(2,))
def _emb(idx, table, B):
    chunks = B // (NW * C)
    mesh = plsc.VectorSubcoreMesh(core_axis_name="c", subcore_axis_name="s")

    @functools.partial(
        pl.kernel,
        out_type=jax.ShapeDtypeStruct((B, D_MODEL), jnp.float32),
        mesh=mesh,
        scratch_types=(
            [pltpu.VMEM((chunks, C), jnp.int32)]
            + [pltpu.VMEM((C, D_MODEL), jnp.float32)] * (NIN + NOUT)
            + [pltpu.SemaphoreType.DMA] * (NIN + NOUT)
        ),
    )
    def emb_kernel(idx_hbm, table_hbm, out_hbm, idx_v, *bufs_and_sems):
        ins = bufs_and_sems[:NIN]
        outs = bufs_and_sems[NIN:NIN + NOUT]
        sis = bufs_and_sems[NIN + NOUT:2 * NIN + NOUT]
        sos = bufs_and_sems[2 * NIN + NOUT:]
        wid = lax.axis_index("s") * NC + lax.axis_index("c")
        base = wid * (chunks * C)
        pltpu.sync_copy(idx_hbm.at[wid], idx_v)
        for b in range(NIN):
            pltpu.async_copy(table_hbm.at[idx_v.at[b]], ins[b], sis[b])

        def outer(jj, carry):
            for u in range(NIN):
                j = NIN * jj + u
                b, ob = u, u % NOUT
                inb, sib = ins[b], sis[b]
                outb, sob = outs[ob], sos[ob]
                pltpu.make_async_copy(table_hbm.at[idx_v.at[j]], inb, sib).wait()

                @pl.when(j >= NOUT)
                def _():
                    pltpu.make_async_copy(
                        outb, out_hbm.at[pl.ds(base, C)], sob).wait()

                @plsc.parallel_loop(0, C, 1)
                def row_body(r):
                    for k in range(D_MODEL // LANES):
                        sl = pl.ds(k * LANES, LANES)
                        outb[r, sl] = inb[r, sl] * SCALE

                @pl.when(j < chunks - NIN)
                def _():
                    pltpu.async_copy(table_hbm.at[idx_v.at[j + NIN]], inb, sib)

                pltpu.async_copy(outb, out_hbm.at[pl.ds(base + j * C, C)], sob)
            return carry

        lax.fori_loop(0, chunks // NIN, outer, 0)
        for u in range(NOUT):
            j = chunks - NOUT + u
            pltpu.make_async_copy(
                outs[j % NOUT], out_hbm.at[pl.ds(base + j * C, C)],
                sos[j % NOUT]).wait()

    return emb_kernel(idx, table)


def kernel(x, table):
    b, s = x.shape
    B = b * s
    idx = x.reshape(NW, B // (NW * C), C).astype(jnp.int32)
    out = _emb(idx, table, B)
    return out.reshape(b, s, D_MODEL)
